# Initial kernel scaffold; baseline (speedup 1.0000x reference)
import jax
import jax.numpy as jnp
from jax.experimental import pallas as pl


def kernel(box_cls, box_regression, image_sizes):
    raise NotImplementedError("write your pallas kernel here")


# speedup vs baseline: 4.5208x; 4.5208x over previous
import jax
import jax.numpy as jnp
from jax import lax
from jax.experimental import pallas as pl
from jax.experimental.pallas import tpu as pltpu

_THRESH = 0.05
_TOPN = 1000
_NMS_T = 0.4
_ROWS = 200
_LANES = 128
_SROWS = 8
_NEG = -1e9


def _pp_kernel(sc_ref, reg_ref, xg_ref, yg_ref,
               osc_ref, obox_ref, olb_ref,
               msc, b0, b1, b2, b3,
               X1, Y1, X2, Y2, KP, SSC):
    s = sc_ref[0]
    msc[...] = jnp.where(s > _THRESH, s, _NEG)
    xg = xg_ref[...]
    yg = yg_ref[...]
    b0[...] = xg - reg_ref[0, 3]
    b1[...] = yg - reg_ref[0, 0]
    b2[...] = xg + reg_ref[0, 1]
    b3[...] = yg + reg_ref[0, 2]

    zero8 = jnp.zeros((_SROWS, _LANES), jnp.float32)
    X1[...] = zero8
    Y1[...] = zero8
    X2[...] = zero8
    Y2[...] = zero8
    KP[...] = zero8
    SSC[...] = zero8

    def _lin(shape):
        return (lax.broadcasted_iota(jnp.int32, shape, 0) * _LANES
                + lax.broadcasted_iota(jnp.int32, shape, 1))

    def ext_body(r, w):
        cur = msc[...]
        lin = _lin((_ROWS, _LANES))
        m = jnp.max(cur)
        p = jnp.min(jnp.where(cur == m, lin, jnp.int32(1 << 30)))
        hit = lin == p
        msc[...] = jnp.where(hit, _NEG, cur)
        x1 = jnp.sum(jnp.where(hit, b0[...], 0.0))
        y1 = jnp.sum(jnp.where(hit, b1[...], 0.0))
        x2 = jnp.sum(jnp.where(hit, b2[...], 0.0))
        y2 = jnp.sum(jnp.where(hit, b3[...], 0.0))
        vflag = (m > _THRESH) & (x2 - x1 >= 0.0) & (y2 - y1 >= 0.0)
        slot = _lin((_SROWS, _LANES))
        smask = (slot == w) & vflag
        X1[...] = jnp.where(smask, x1, X1[...])
        Y1[...] = jnp.where(smask, y1, Y1[...])
        X2[...] = jnp.where(smask, x2, X2[...])
        Y2[...] = jnp.where(smask, y2, Y2[...])
        SSC[...] = jnp.where(smask, m, SSC[...])
        KP[...] = jnp.where(smask, 1.0, KP[...])
        return w + vflag.astype(jnp.int32)

    lax.fori_loop(0, _TOPN, ext_body, jnp.int32(0), unroll=False)

    def nms_body(i, carry):
        slot = _lin((_SROWS, _LANES))
        sel = slot == i
        kp = KP[...]
        x1p = X1[...]
        y1p = Y1[...]
        x2p = X2[...]
        y2p = Y2[...]
        cur = jnp.sum(jnp.where(sel, kp, 0.0))
        x1i = jnp.sum(jnp.where(sel, x1p, 0.0))
        y1i = jnp.sum(jnp.where(sel, y1p, 0.0))
        x2i = jnp.sum(jnp.where(sel, x2p, 0.0))
        y2i = jnp.sum(jnp.where(sel, y2p, 0.0))
        ai = jnp.maximum(x2i - x1i, 0.0) * jnp.maximum(y2i - y1i, 0.0)
        areas = jnp.maximum(x2p - x1p, 0.0) * jnp.maximum(y2p - y1p, 0.0)
        inter = (jnp.maximum(jnp.minimum(x2p, x2i) - jnp.maximum(x1p, x1i), 0.0)
                 * jnp.maximum(jnp.minimum(y2p, y2i) - jnp.maximum(y1p, y1i), 0.0))
        union = ai + areas - inter
        iou = inter / jnp.maximum(union, 1e-9)
        sup = (iou > _NMS_T) & (slot > i) & (cur > 0.0)
        KP[...] = jnp.where(sup, 0.0, kp)
        return carry

    lax.fori_loop(0, _TOPN, nms_body, jnp.int32(0), unroll=False)

    kp = KP[...]
    keepb = kp > 0.0
    osc_ref[0] = jnp.where(keepb, SSC[...], 0.0)
    olb_ref[0] = keepb.astype(jnp.int32)
    obox_ref[0, 0] = jnp.where(keepb, X1[...], 0.0)
    obox_ref[0, 1] = jnp.where(keepb, Y1[...], 0.0)
    obox_ref[0, 2] = jnp.where(keepb, X2[...], 0.0)
    obox_ref[0, 3] = jnp.where(keepb, Y2[...], 0.0)


@jax.jit
def kernel(box_cls, box_regression, image_sizes):
    N, C, H, W = box_cls.shape
    scores = box_cls.reshape(N, _ROWS, _LANES)
    reg = box_regression.reshape(N, 4, _ROWS, _LANES)
    xs = jnp.broadcast_to(jnp.arange(W, dtype=jnp.float32)[None, :] * 4.0,
                          (H, W)).reshape(_ROWS, _LANES)
    ys = jnp.broadcast_to(jnp.arange(H, dtype=jnp.float32)[:, None] * 4.0,
                          (H, W)).reshape(_ROWS, _LANES)

    out_shape = (
        jax.ShapeDtypeStruct((N, _SROWS, _LANES), jnp.float32),
        jax.ShapeDtypeStruct((N, 4, _SROWS, _LANES), jnp.float32),
        jax.ShapeDtypeStruct((N, _SROWS, _LANES), jnp.int32),
    )
    full = lambda i: (0, 0)
    sc8, box8, lb8 = pl.pallas_call(
        _pp_kernel,
        grid=(N,),
        in_specs=[
            pl.BlockSpec((1, _ROWS, _LANES), lambda i: (i, 0, 0)),
            pl.BlockSpec((1, 4, _ROWS, _LANES), lambda i: (i, 0, 0, 0)),
            pl.BlockSpec((_ROWS, _LANES), full),
            pl.BlockSpec((_ROWS, _LANES), full),
        ],
        out_specs=(
            pl.BlockSpec((1, _SROWS, _LANES), lambda i: (i, 0, 0)),
            pl.BlockSpec((1, 4, _SROWS, _LANES), lambda i: (i, 0, 0, 0)),
            pl.BlockSpec((1, _SROWS, _LANES), lambda i: (i, 0, 0)),
        ),
        out_shape=out_shape,
        scratch_shapes=[pltpu.VMEM((_ROWS, _LANES), jnp.float32)] * 5
        + [pltpu.VMEM((_SROWS, _LANES), jnp.float32)] * 6,
    )(scores, reg, xs, ys)

    scores_out = sc8.reshape(N, _SROWS * _LANES)[:, :_TOPN]
    boxes_out = jnp.transpose(box8.reshape(N, 4, _SROWS * _LANES),
                              (0, 2, 1))[:, :_TOPN]
    labels_out = lb8.reshape(N, _SROWS * _LANES)[:, :_TOPN]
    return boxes_out, scores_out, labels_out


# speedup vs baseline: 4.8695x; 1.0771x over previous
import jax
import jax.numpy as jnp
from jax import lax
from jax.experimental import pallas as pl
from jax.experimental.pallas import tpu as pltpu

_THRESH = 0.05
_TOPN = 1000
_NMS_T = 0.4
_ROWS = 200
_LANES = 128
_SROWS = 8
_NEG = -1e9


def _pp_kernel(sc_ref, reg_ref, xg_ref, yg_ref,
               osc_ref, obox_ref, olb_ref,
               msc, b0, b1, b2, b3,
               X1, Y1, X2, Y2, KP, SSC):
    s = sc_ref[0]
    msc[...] = jnp.where(s > _THRESH, s, _NEG)
    xg = xg_ref[...]
    yg = yg_ref[...]
    b0[...] = xg - reg_ref[0, 3]
    b1[...] = yg - reg_ref[0, 0]
    b2[...] = xg + reg_ref[0, 1]
    b3[...] = yg + reg_ref[0, 2]

    zero8 = jnp.zeros((_SROWS, _LANES), jnp.float32)
    X1[...] = zero8
    Y1[...] = zero8
    X2[...] = zero8
    Y2[...] = zero8
    KP[...] = zero8
    SSC[...] = zero8

    def _lin(shape):
        return (lax.broadcasted_iota(jnp.int32, shape, 0) * _LANES
                + lax.broadcasted_iota(jnp.int32, shape, 1))

    lane = lax.broadcasted_iota(jnp.int32, (1, _LANES), 1)

    def ext_body(r, w):
        cur = msc[...]
        lin = _lin((_ROWS, _LANES))
        m = jnp.max(cur)
        p = jnp.min(jnp.where(cur == m, lin, jnp.int32(1 << 30)))
        rr = p // _LANES
        cc = p - rr * _LANES
        hit1 = lane == cc
        srow = msc[pl.ds(rr, 1), :]
        msc[pl.ds(rr, 1), :] = jnp.where(hit1, _NEG, srow)
        x1 = jnp.sum(jnp.where(hit1, b0[pl.ds(rr, 1), :], 0.0))
        y1 = jnp.sum(jnp.where(hit1, b1[pl.ds(rr, 1), :], 0.0))
        x2 = jnp.sum(jnp.where(hit1, b2[pl.ds(rr, 1), :], 0.0))
        y2 = jnp.sum(jnp.where(hit1, b3[pl.ds(rr, 1), :], 0.0))
        vflag = (m > _THRESH) & (x2 - x1 >= 0.0) & (y2 - y1 >= 0.0)
        slot = _lin((_SROWS, _LANES))
        smask = (slot == w) & vflag
        X1[...] = jnp.where(smask, x1, X1[...])
        Y1[...] = jnp.where(smask, y1, Y1[...])
        X2[...] = jnp.where(smask, x2, X2[...])
        Y2[...] = jnp.where(smask, y2, Y2[...])
        SSC[...] = jnp.where(smask, m, SSC[...])
        KP[...] = jnp.where(smask, 1.0, KP[...])
        return w + vflag.astype(jnp.int32)

    lax.fori_loop(0, _TOPN, ext_body, jnp.int32(0), unroll=False)

    def nms_body(i, carry):
        slot = _lin((_SROWS, _LANES))
        sel = slot == i
        kp = KP[...]
        x1p = X1[...]
        y1p = Y1[...]
        x2p = X2[...]
        y2p = Y2[...]
        cur = jnp.sum(jnp.where(sel, kp, 0.0))
        x1i = jnp.sum(jnp.where(sel, x1p, 0.0))
        y1i = jnp.sum(jnp.where(sel, y1p, 0.0))
        x2i = jnp.sum(jnp.where(sel, x2p, 0.0))
        y2i = jnp.sum(jnp.where(sel, y2p, 0.0))
        ai = jnp.maximum(x2i - x1i, 0.0) * jnp.maximum(y2i - y1i, 0.0)
        areas = jnp.maximum(x2p - x1p, 0.0) * jnp.maximum(y2p - y1p, 0.0)
        inter = (jnp.maximum(jnp.minimum(x2p, x2i) - jnp.maximum(x1p, x1i), 0.0)
                 * jnp.maximum(jnp.minimum(y2p, y2i) - jnp.maximum(y1p, y1i), 0.0))
        union = ai + areas - inter
        iou = inter / jnp.maximum(union, 1e-9)
        sup = (iou > _NMS_T) & (slot > i) & (cur > 0.0)
        KP[...] = jnp.where(sup, 0.0, kp)
        return carry

    lax.fori_loop(0, _TOPN, nms_body, jnp.int32(0), unroll=False)

    kp = KP[...]
    keepb = kp > 0.0
    osc_ref[0] = jnp.where(keepb, SSC[...], 0.0)
    olb_ref[0] = keepb.astype(jnp.int32)
    obox_ref[0, 0] = jnp.where(keepb, X1[...], 0.0)
    obox_ref[0, 1] = jnp.where(keepb, Y1[...], 0.0)
    obox_ref[0, 2] = jnp.where(keepb, X2[...], 0.0)
    obox_ref[0, 3] = jnp.where(keepb, Y2[...], 0.0)


@jax.jit
def kernel(box_cls, box_regression, image_sizes):
    N, C, H, W = box_cls.shape
    scores = box_cls.reshape(N, _ROWS, _LANES)
    reg = box_regression.reshape(N, 4, _ROWS, _LANES)
    xs = jnp.broadcast_to(jnp.arange(W, dtype=jnp.float32)[None, :] * 4.0,
                          (H, W)).reshape(_ROWS, _LANES)
    ys = jnp.broadcast_to(jnp.arange(H, dtype=jnp.float32)[:, None] * 4.0,
                          (H, W)).reshape(_ROWS, _LANES)

    out_shape = (
        jax.ShapeDtypeStruct((N, _SROWS, _LANES), jnp.float32),
        jax.ShapeDtypeStruct((N, 4, _SROWS, _LANES), jnp.float32),
        jax.ShapeDtypeStruct((N, _SROWS, _LANES), jnp.int32),
    )
    full = lambda i: (0, 0)
    sc8, box8, lb8 = pl.pallas_call(
        _pp_kernel,
        grid=(N,),
        in_specs=[
            pl.BlockSpec((1, _ROWS, _LANES), lambda i: (i, 0, 0)),
            pl.BlockSpec((1, 4, _ROWS, _LANES), lambda i: (i, 0, 0, 0)),
            pl.BlockSpec((_ROWS, _LANES), full),
            pl.BlockSpec((_ROWS, _LANES), full),
        ],
        out_specs=(
            pl.BlockSpec((1, _SROWS, _LANES), lambda i: (i, 0, 0)),
            pl.BlockSpec((1, 4, _SROWS, _LANES), lambda i: (i, 0, 0, 0)),
            pl.BlockSpec((1, _SROWS, _LANES), lambda i: (i, 0, 0)),
        ),
        out_shape=out_shape,
        scratch_shapes=[pltpu.VMEM((_ROWS, _LANES), jnp.float32)] * 5
        + [pltpu.VMEM((_SROWS, _LANES), jnp.float32)] * 6,
    )(scores, reg, xs, ys)

    scores_out = sc8.reshape(N, _SROWS * _LANES)[:, :_TOPN]
    boxes_out = jnp.transpose(box8.reshape(N, 4, _SROWS * _LANES),
                              (0, 2, 1))[:, :_TOPN]
    labels_out = lb8.reshape(N, _SROWS * _LANES)[:, :_TOPN]
    return boxes_out, scores_out, labels_out


# speedup vs baseline: 4.8725x; 1.0006x over previous
import jax
import jax.numpy as jnp
from jax import lax
from jax.experimental import pallas as pl
from jax.experimental.pallas import tpu as pltpu

_THRESH = 0.05
_TOPN = 1000
_NMS_T = 0.4
_ROWS = 200
_LANES = 128
_SROWS = 8
_NEG = -1e9


def _pp_kernel(sc_ref, reg_ref, xg_ref, yg_ref,
               osc_ref, obox_ref, olb_ref,
               msc, b0, b1, b2, b3,
               X1, Y1, X2, Y2, KP, SSC):
    s = sc_ref[0]
    msc[...] = jnp.where(s > _THRESH, s, _NEG)
    xg = xg_ref[...]
    yg = yg_ref[...]
    b0[...] = xg - reg_ref[0, 3]
    b1[...] = yg - reg_ref[0, 0]
    b2[...] = xg + reg_ref[0, 1]
    b3[...] = yg + reg_ref[0, 2]

    zero8 = jnp.zeros((_SROWS, _LANES), jnp.float32)
    X1[...] = zero8
    Y1[...] = zero8
    X2[...] = zero8
    Y2[...] = zero8
    KP[...] = zero8
    SSC[...] = zero8

    def _lin(shape):
        return (lax.broadcasted_iota(jnp.int32, shape, 0) * _LANES
                + lax.broadcasted_iota(jnp.int32, shape, 1))

    lane = lax.broadcasted_iota(jnp.int32, (1, _LANES), 1)

    def ext_body(r, w):
        cur = msc[...]
        lin = _lin((_ROWS, _LANES))
        m = jnp.max(cur)
        p = jnp.min(jnp.where(cur == m, lin, jnp.int32(1 << 30)))
        rr = p // _LANES
        cc = p - rr * _LANES
        hit1 = lane == cc
        srow = msc[pl.ds(rr, 1), :]
        msc[pl.ds(rr, 1), :] = jnp.where(hit1, _NEG, srow)
        x1 = jnp.sum(jnp.where(hit1, b0[pl.ds(rr, 1), :], 0.0))
        y1 = jnp.sum(jnp.where(hit1, b1[pl.ds(rr, 1), :], 0.0))
        x2 = jnp.sum(jnp.where(hit1, b2[pl.ds(rr, 1), :], 0.0))
        y2 = jnp.sum(jnp.where(hit1, b3[pl.ds(rr, 1), :], 0.0))
        vflag = (m > _THRESH) & (x2 - x1 >= 0.0) & (y2 - y1 >= 0.0)
        slot = _lin((_SROWS, _LANES))
        smask = (slot == w) & vflag
        X1[...] = jnp.where(smask, x1, X1[...])
        Y1[...] = jnp.where(smask, y1, Y1[...])
        X2[...] = jnp.where(smask, x2, X2[...])
        Y2[...] = jnp.where(smask, y2, Y2[...])
        SSC[...] = jnp.where(smask, m, SSC[...])
        KP[...] = jnp.where(smask, 1.0, KP[...])
        return w + vflag.astype(jnp.int32)

    lax.fori_loop(0, _TOPN, ext_body, jnp.int32(0), unroll=False)

    def nms_body(i, carry):
        slot = _lin((_SROWS, _LANES))
        sel = slot == i
        kp = KP[...]
        x1p = X1[...]
        y1p = Y1[...]
        x2p = X2[...]
        y2p = Y2[...]
        cur = jnp.sum(jnp.where(sel, kp, 0.0))
        x1i = jnp.sum(jnp.where(sel, x1p, 0.0))
        y1i = jnp.sum(jnp.where(sel, y1p, 0.0))
        x2i = jnp.sum(jnp.where(sel, x2p, 0.0))
        y2i = jnp.sum(jnp.where(sel, y2p, 0.0))
        ai = jnp.maximum(x2i - x1i, 0.0) * jnp.maximum(y2i - y1i, 0.0)
        areas = jnp.maximum(x2p - x1p, 0.0) * jnp.maximum(y2p - y1p, 0.0)
        inter = (jnp.maximum(jnp.minimum(x2p, x2i) - jnp.maximum(x1p, x1i), 0.0)
                 * jnp.maximum(jnp.minimum(y2p, y2i) - jnp.maximum(y1p, y1i), 0.0))
        union = ai + areas - inter
        iou = inter / jnp.maximum(union, 1e-9)
        sup = (iou > _NMS_T) & (slot > i) & (cur > 0.0)
        KP[...] = jnp.where(sup, 0.0, kp)
        return carry

    lax.fori_loop(0, _TOPN, nms_body, jnp.int32(0), unroll=False)

    kp = KP[...]
    keepb = kp > 0.0
    osc_ref[0] = jnp.where(keepb, SSC[...], 0.0)
    olb_ref[0] = keepb.astype(jnp.int32)
    obox_ref[0, 0] = jnp.where(keepb, X1[...], 0.0)
    obox_ref[0, 1] = jnp.where(keepb, Y1[...], 0.0)
    obox_ref[0, 2] = jnp.where(keepb, X2[...], 0.0)
    obox_ref[0, 3] = jnp.where(keepb, Y2[...], 0.0)


@jax.jit
def kernel(box_cls, box_regression, image_sizes):
    N, C, H, W = box_cls.shape
    scores = box_cls.reshape(N, _ROWS, _LANES)
    reg = box_regression.reshape(N, 4, _ROWS, _LANES)
    xs = jnp.broadcast_to(jnp.arange(W, dtype=jnp.float32)[None, :] * 4.0,
                          (H, W)).reshape(_ROWS, _LANES)
    ys = jnp.broadcast_to(jnp.arange(H, dtype=jnp.float32)[:, None] * 4.0,
                          (H, W)).reshape(_ROWS, _LANES)

    out_shape = (
        jax.ShapeDtypeStruct((N, _SROWS, _LANES), jnp.float32),
        jax.ShapeDtypeStruct((N, 4, _SROWS, _LANES), jnp.float32),
        jax.ShapeDtypeStruct((N, _SROWS, _LANES), jnp.int32),
    )
    full = lambda i: (0, 0)
    sc8, box8, lb8 = pl.pallas_call(
        _pp_kernel,
        grid=(N,),
        in_specs=[
            pl.BlockSpec((1, _ROWS, _LANES), lambda i: (i, 0, 0)),
            pl.BlockSpec((1, 4, _ROWS, _LANES), lambda i: (i, 0, 0, 0)),
            pl.BlockSpec((_ROWS, _LANES), full),
            pl.BlockSpec((_ROWS, _LANES), full),
        ],
        out_specs=(
            pl.BlockSpec((1, _SROWS, _LANES), lambda i: (i, 0, 0)),
            pl.BlockSpec((1, 4, _SROWS, _LANES), lambda i: (i, 0, 0, 0)),
            pl.BlockSpec((1, _SROWS, _LANES), lambda i: (i, 0, 0)),
        ),
        out_shape=out_shape,
        scratch_shapes=[pltpu.VMEM((_ROWS, _LANES), jnp.float32)] * 5
        + [pltpu.VMEM((_SROWS, _LANES), jnp.float32)] * 6,
        compiler_params=pltpu.CompilerParams(
            dimension_semantics=("parallel",)),
    )(scores, reg, xs, ys)

    scores_out = sc8.reshape(N, _SROWS * _LANES)[:, :_TOPN]
    boxes_out = jnp.transpose(box8.reshape(N, 4, _SROWS * _LANES),
                              (0, 2, 1))[:, :_TOPN]
    labels_out = lb8.reshape(N, _SROWS * _LANES)[:, :_TOPN]
    return boxes_out, scores_out, labels_out
